# baseline (speedup 1.0000x reference)
import functools

import jax
import jax.numpy as jnp
from jax import lax
from jax.experimental import pallas as pl
from jax.experimental.pallas import tpu as pltpu
from jax.experimental.pallas import tpu_sc as plsc

_POS = 0.5
_NEG = 0.3
_LOW = 0.1
_N_PAD = 20480
_NW = 32
_PER_W = _N_PAD // _NW
_CHUNKS = _PER_W // 16
_G = 64
_BIG = 2**30


def _lane():
    return lax.broadcasted_iota(jnp.int32, (16,), 0)


def _bcast(x, l):
    idx = jnp.full((16,), l, jnp.int32)
    return x.at[idx].get(mode="promise_in_bounds")


def _bmax(x):
    for k in (1, 2, 4, 8):
        x = jnp.maximum(x, x.at[_lane() ^ k].get(mode="promise_in_bounds"))
    return x


def _bmin(x):
    for k in (1, 2, 4, 8):
        x = jnp.minimum(x, x.at[_lane() ^ k].get(mode="promise_in_bounds"))
    return x


def _wid():
    return lax.axis_index("s") * 2 + lax.axis_index("c")


def _phase1(ax0, ay0, ax1, ay1, gb0, gb1, gb2, gb3,
            assign_out, candv_out, candi_out,
            a0r, a1r, a2r, a3r, g0r, g1r, g2r, g3r,
            rmr, rir, asgr, cvr, cir):
    wid = _wid()
    base = wid * _PER_W
    pltpu.sync_copy(ax0.at[pl.ds(base, _PER_W)], a0r)
    pltpu.sync_copy(ay0.at[pl.ds(base, _PER_W)], a1r)
    pltpu.sync_copy(ax1.at[pl.ds(base, _PER_W)], a2r)
    pltpu.sync_copy(ay1.at[pl.ds(base, _PER_W)], a3r)
    pltpu.sync_copy(gb0, g0r)
    pltpu.sync_copy(gb1, g1r)
    pltpu.sync_copy(gb2, g2r)
    pltpu.sync_copy(gb3, g3r)

    lane = _lane()
    neg1f = jnp.full((16,), -1.0, jnp.float32)
    zeroi = jnp.full((16,), 0, jnp.int32)

    def init_c(c, _):
        rmr[pl.ds(c * 16, 16)] = neg1f
        rir[pl.ds(c * 16, 16)] = zeroi
        return 0
    lax.fori_loop(0, _CHUNKS, init_c, 0)

    def per_gt(g, _):
        goff = g * 16
        g0 = g0r[pl.ds(goff, 16)]
        g1 = g1r[pl.ds(goff, 16)]
        g2 = g2r[pl.ds(goff, 16)]
        g3 = g3r[pl.ds(goff, 16)]
        area_b = (g2 - g0) * (g3 - g1)
        gvec = jnp.full((16,), g, jnp.int32)

        def per_chunk(c, carry):
            t1, t2, t3, j1, j2, j3 = carry
            off = c * 16
            a0 = a0r[pl.ds(off, 16)]
            a1 = a1r[pl.ds(off, 16)]
            a2 = a2r[pl.ds(off, 16)]
            a3 = a3r[pl.ds(off, 16)]
            area_a = (a2 - a0) * (a3 - a1)
            ltx = jnp.maximum(a0, g0)
            lty = jnp.maximum(a1, g1)
            rbx = jnp.minimum(a2, g2)
            rby = jnp.minimum(a3, g3)
            w = jnp.maximum(rbx - ltx, 0.0)
            h = jnp.maximum(rby - lty, 0.0)
            inter = w * h
            union = area_a + area_b - inter
            iou = inter / jnp.maximum(union, 1e-9)

            rm = rmr[pl.ds(off, 16)]
            upd = iou > rm
            rmr[pl.ds(off, 16)] = jnp.where(upd, iou, rm)
            rir[pl.ds(off, 16)] = jnp.where(upd, gvec, rir[pl.ds(off, 16)])

            gidx = (base + off) + lane
            b1 = iou > t1
            b2 = iou > t2
            b3 = iou > t3
            t3n = jnp.where(b2, t2, jnp.where(b3, iou, t3))
            j3n = jnp.where(b2, j2, jnp.where(b3, gidx, j3))
            t2n = jnp.where(b1, t1, jnp.where(b2, iou, t2))
            j2n = jnp.where(b1, j1, jnp.where(b2, gidx, j2))
            t1n = jnp.where(b1, iou, t1)
            j1n = jnp.where(b1, gidx, j1)
            return t1n, t2n, t3n, j1n, j2n, j3n

        init = (neg1f, neg1f, neg1f, zeroi, zeroi, zeroi)
        t1, t2, t3, j1, j2, j3 = lax.fori_loop(0, _CHUNKS, per_chunk, init)

        d = zeroi
        m1 = _bmax(t1)
        i1 = _bmin(jnp.where(t1 == m1, j1, _BIG))
        d = d + jnp.where((t1 == m1) & (j1 == i1), 1, 0)
        ev = jnp.where(d == 1, t2, t1)
        ej = jnp.where(d == 1, j2, j1)
        m2 = _bmax(ev)
        i2 = _bmin(jnp.where(ev == m2, ej, _BIG))
        d = d + jnp.where((ev == m2) & (ej == i2), 1, 0)
        ev = jnp.where(d == 2, t3, jnp.where(d == 1, t2, t1))
        ej = jnp.where(d == 2, j3, jnp.where(d == 1, j2, j1))
        m3 = _bmax(ev)
        i3 = _bmin(jnp.where(ev == m3, ej, _BIG))

        cvr[pl.ds(goff, 16)] = jnp.where(
            lane == 0, m1, jnp.where(lane == 1, m2,
                                     jnp.where(lane == 2, m3, -1.0)))
        cir[pl.ds(goff, 16)] = jnp.where(
            lane == 0, i1, jnp.where(lane == 1, i2,
                                     jnp.where(lane == 2, i3, _BIG)))
        return 0

    lax.fori_loop(0, _G, per_gt, 0)

    def base_c(c, _):
        off = c * 16
        rm = rmr[pl.ds(off, 16)]
        a = jnp.where(rm > _POS, rir[pl.ds(off, 16)], -2)
        a = jnp.where(rm < _NEG, -1, a)
        asgr[pl.ds(off, 16)] = a
        return 0
    lax.fori_loop(0, _CHUNKS, base_c, 0)

    pltpu.sync_copy(asgr, assign_out.at[pl.ds(base, _PER_W)])
    pltpu.sync_copy(cvr, candv_out.at[pl.ds(wid * _G * 16, _G * 16)])
    pltpu.sync_copy(cir, candi_out.at[pl.ds(wid * _G * 16, _G * 16)])


def _phase2(candv, candi, writes_out, cva, cia, wvr):
    wid = _wid()
    lane = _lane()
    pltpu.sync_copy(candv, cva)
    pltpu.sync_copy(candi, cia)

    for t in range(2):
        g = wid * 2 + t
        goff = g * 16

        ms = []
        is_ = []
        for _r in range(3):
            def fold_max(w, run):
                return jnp.maximum(run, cva[pl.ds(w * (_G * 16) + goff, 16)])
            runm = lax.fori_loop(0, _NW, fold_max,
                                 jnp.full((16,), -1.0, jnp.float32))
            mv = _bmax(runm)

            def fold_idx(w, run):
                off = w * (_G * 16) + goff
                return jnp.minimum(
                    run, jnp.where(cva[pl.ds(off, 16)] == mv,
                                   cia[pl.ds(off, 16)], _BIG))
            runi = lax.fori_loop(0, _NW, fold_idx,
                                 jnp.full((16,), _BIG, jnp.int32))
            iv = _bmin(runi)

            def mask_out(w, _):
                off = w * (_G * 16) + goff
                cva[pl.ds(off, 16)] = jnp.where(
                    cia[pl.ds(off, 16)] == iv, -1.0, cva[pl.ds(off, 16)])
                return 0
            lax.fori_loop(0, _NW, mask_out, 0)
            ms.append(mv)
            is_.append(iv)

        m1, m2, m3 = ms
        i1, i2, i3 = is_
        nposv = (jnp.where(m1 > _POS, 1, 0) + jnp.where(m2 > _POS, 1, 0)
                 + jnp.where(m3 > _POS, 1, 0))
        condv = nposv < 3
        l2 = jnp.where((m2 > _LOW) & condv, i2, -1)
        l3 = jnp.where((m3 > _LOW) & condv, i3, -1)
        wvr[...] = jnp.where(lane == 0, i1,
                             jnp.where(lane == 1, l2,
                                       jnp.where(lane == 2, l3, -1)))
        pltpu.sync_copy(wvr, writes_out.at[pl.ds(goff, 16)])


def _phase3(assign, writes, final_out, asgr, wvr):
    wid = _wid()
    base = wid * _PER_W
    lane = _lane()
    pltpu.sync_copy(assign.at[pl.ds(base, _PER_W)], asgr)
    pltpu.sync_copy(writes, wvr)

    def per_gt(g, _):
        wv = wvr[pl.ds(g * 16, 16)]
        t0 = _bcast(wv, 0)
        t1 = _bcast(wv, 1)
        t2 = _bcast(wv, 2)
        gvec = jnp.full((16,), g, jnp.int32)

        def per_chunk(c, _c):
            off = c * 16
            a = asgr[pl.ds(off, 16)]
            gidx = (base + off) + lane
            upd = (gidx == t0) | (gidx == t1) | (gidx == t2)
            asgr[pl.ds(off, 16)] = jnp.where(upd, gvec, a)
            return 0
        lax.fori_loop(0, _CHUNKS, per_chunk, 0)
        return 0
    lax.fori_loop(0, _G, per_gt, 0)

    pltpu.sync_copy(asgr, final_out.at[pl.ds(base, _PER_W)])


@functools.lru_cache(maxsize=1)
def _build():
    mesh = plsc.VectorSubcoreMesh(core_axis_name="c", subcore_axis_name="s")
    f32, i32 = jnp.float32, jnp.int32
    p1 = pl.kernel(
        _phase1, mesh=mesh,
        out_type=(
            jax.ShapeDtypeStruct((_N_PAD,), i32),
            jax.ShapeDtypeStruct((_NW * _G * 16,), f32),
            jax.ShapeDtypeStruct((_NW * _G * 16,), i32),
        ),
        scratch_types=[
            pltpu.VMEM((_PER_W,), f32),
            pltpu.VMEM((_PER_W,), f32),
            pltpu.VMEM((_PER_W,), f32),
            pltpu.VMEM((_PER_W,), f32),
            pltpu.VMEM((_G * 16,), f32),
            pltpu.VMEM((_G * 16,), f32),
            pltpu.VMEM((_G * 16,), f32),
            pltpu.VMEM((_G * 16,), f32),
            pltpu.VMEM((_PER_W,), f32),
            pltpu.VMEM((_PER_W,), i32),
            pltpu.VMEM((_PER_W,), i32),
            pltpu.VMEM((_G * 16,), f32),
            pltpu.VMEM((_G * 16,), i32),
        ],
    )
    p2 = pl.kernel(
        _phase2, mesh=mesh,
        out_type=jax.ShapeDtypeStruct((_G * 16,), i32),
        scratch_types=[
            pltpu.VMEM((_NW * _G * 16,), f32),
            pltpu.VMEM((_NW * _G * 16,), i32),
            pltpu.VMEM((16,), i32),
        ],
    )
    p3 = pl.kernel(
        _phase3, mesh=mesh,
        out_type=jax.ShapeDtypeStruct((_N_PAD,), i32),
        scratch_types=[
            pltpu.VMEM((_PER_W,), i32),
            pltpu.VMEM((_G * 16,), i32),
        ],
    )
    return p1, p2, p3


def kernel(anchor, gt):
    n = anchor.shape[0]
    a = jnp.pad(anchor, ((0, _N_PAD - n), (0, 0)))
    gb = [jnp.repeat(gt[:, c], 16) for c in range(4)]
    p1, p2, p3 = _build()
    assign, candv, candi = p1(a[:, 0], a[:, 1], a[:, 2], a[:, 3], *gb)
    writes = p2(candv, candi)
    final = p3(assign, writes)
    return final[:n]


# speedup vs baseline: 1.0919x; 1.0919x over previous
import functools

import jax
import jax.numpy as jnp
from jax import lax
from jax.experimental import pallas as pl
from jax.experimental.pallas import tpu as pltpu
from jax.experimental.pallas import tpu_sc as plsc

_POS = 0.5
_NEG = 0.3
_LOW = 0.1
_N_PAD = 20480
_NW = 32
_PER_W = _N_PAD // _NW
_CHUNKS = _PER_W // 16
_G = 64
_BIG = 2**30


def _lane():
    return lax.broadcasted_iota(jnp.int32, (16,), 0)


def _bcast(x, l):
    idx = jnp.full((16,), l, jnp.int32)
    return x.at[idx].get(mode="promise_in_bounds")


def _bmax(x):
    for k in (1, 2, 4, 8):
        x = jnp.maximum(x, x.at[_lane() ^ k].get(mode="promise_in_bounds"))
    return x


def _bmin(x):
    for k in (1, 2, 4, 8):
        x = jnp.minimum(x, x.at[_lane() ^ k].get(mode="promise_in_bounds"))
    return x


def _wid():
    return lax.axis_index("s") * 2 + lax.axis_index("c")


def _phase1(ax0, ay0, ax1, ay1, gb0, gb1, gb2, gb3,
            assign_out, candv_out, candi_out,
            a0r, a1r, a2r, a3r, g0r, g1r, g2r, g3r,
            rmr, rir, asgr, cvr, cir):
    wid = _wid()
    base = wid * _PER_W
    pltpu.sync_copy(ax0.at[pl.ds(base, _PER_W)], a0r)
    pltpu.sync_copy(ay0.at[pl.ds(base, _PER_W)], a1r)
    pltpu.sync_copy(ax1.at[pl.ds(base, _PER_W)], a2r)
    pltpu.sync_copy(ay1.at[pl.ds(base, _PER_W)], a3r)
    pltpu.sync_copy(gb0, g0r)
    pltpu.sync_copy(gb1, g1r)
    pltpu.sync_copy(gb2, g2r)
    pltpu.sync_copy(gb3, g3r)

    lane = _lane()
    neg1f = jnp.full((16,), -1.0, jnp.float32)
    zeroi = jnp.full((16,), 0, jnp.int32)

    def init_c(c, _):
        rmr[pl.ds(c * 16, 16)] = neg1f
        rir[pl.ds(c * 16, 16)] = zeroi
        return 0
    lax.fori_loop(0, _CHUNKS, init_c, 0)

    def per_gt(g, _):
        goff = g * 16
        g0 = g0r[pl.ds(goff, 16)]
        g1 = g1r[pl.ds(goff, 16)]
        g2 = g2r[pl.ds(goff, 16)]
        g3 = g3r[pl.ds(goff, 16)]
        area_b = (g2 - g0) * (g3 - g1)
        gvec = jnp.full((16,), g, jnp.int32)

        def per_chunk4(c4, carry):
            t1, t2, t3, j1, j2, j3 = carry
            ious = []
            for u in range(4):
                off = c4 * 64 + u * 16
                a0 = a0r[pl.ds(off, 16)]
                a1 = a1r[pl.ds(off, 16)]
                a2 = a2r[pl.ds(off, 16)]
                a3 = a3r[pl.ds(off, 16)]
                area_a = (a2 - a0) * (a3 - a1)
                ltx = jnp.maximum(a0, g0)
                lty = jnp.maximum(a1, g1)
                rbx = jnp.minimum(a2, g2)
                rby = jnp.minimum(a3, g3)
                w = jnp.maximum(rbx - ltx, 0.0)
                h = jnp.maximum(rby - lty, 0.0)
                inter = w * h
                union = area_a + area_b - inter
                iou = inter / jnp.maximum(union, 1e-9)
                ious.append(iou)

                rm = rmr[pl.ds(off, 16)]
                upd = iou > rm
                rmr[pl.ds(off, 16)] = jnp.where(upd, iou, rm)
                rir[pl.ds(off, 16)] = jnp.where(upd, gvec, rir[pl.ds(off, 16)])

            for u in range(4):
                off = c4 * 64 + u * 16
                iou = ious[u]
                gidx = (base + off) + lane
                b1 = iou > t1
                b2 = iou > t2
                b3 = iou > t3
                t3n = jnp.where(b2, t2, jnp.where(b3, iou, t3))
                j3n = jnp.where(b2, j2, jnp.where(b3, gidx, j3))
                t2n = jnp.where(b1, t1, jnp.where(b2, iou, t2))
                j2n = jnp.where(b1, j1, jnp.where(b2, gidx, j2))
                t1, t2, t3 = jnp.where(b1, iou, t1), t2n, t3n
                j1, j2, j3 = jnp.where(b1, gidx, j1), j2n, j3n
            return t1, t2, t3, j1, j2, j3

        init = (neg1f, neg1f, neg1f, zeroi, zeroi, zeroi)
        t1, t2, t3, j1, j2, j3 = lax.fori_loop(0, _CHUNKS // 4, per_chunk4,
                                               init)

        d = zeroi
        m1 = _bmax(t1)
        i1 = _bmin(jnp.where(t1 == m1, j1, _BIG))
        d = d + jnp.where((t1 == m1) & (j1 == i1), 1, 0)
        ev = jnp.where(d == 1, t2, t1)
        ej = jnp.where(d == 1, j2, j1)
        m2 = _bmax(ev)
        i2 = _bmin(jnp.where(ev == m2, ej, _BIG))
        d = d + jnp.where((ev == m2) & (ej == i2), 1, 0)
        ev = jnp.where(d == 2, t3, jnp.where(d == 1, t2, t1))
        ej = jnp.where(d == 2, j3, jnp.where(d == 1, j2, j1))
        m3 = _bmax(ev)
        i3 = _bmin(jnp.where(ev == m3, ej, _BIG))

        cvr[pl.ds(goff, 16)] = jnp.where(
            lane == 0, m1, jnp.where(lane == 1, m2,
                                     jnp.where(lane == 2, m3, -1.0)))
        cir[pl.ds(goff, 16)] = jnp.where(
            lane == 0, i1, jnp.where(lane == 1, i2,
                                     jnp.where(lane == 2, i3, _BIG)))
        return 0

    lax.fori_loop(0, _G, per_gt, 0)

    def base_c(c, _):
        off = c * 16
        rm = rmr[pl.ds(off, 16)]
        a = jnp.where(rm > _POS, rir[pl.ds(off, 16)], -2)
        a = jnp.where(rm < _NEG, -1, a)
        asgr[pl.ds(off, 16)] = a
        return 0
    lax.fori_loop(0, _CHUNKS, base_c, 0)

    pltpu.sync_copy(asgr, assign_out.at[pl.ds(base, _PER_W)])
    pltpu.sync_copy(cvr, candv_out.at[pl.ds(wid * _G * 16, _G * 16)])
    pltpu.sync_copy(cir, candi_out.at[pl.ds(wid * _G * 16, _G * 16)])


def _phase2(candv, candi, writes_out, cva, cia, wvr):
    wid = _wid()
    lane = _lane()
    pltpu.sync_copy(candv, cva)
    pltpu.sync_copy(candi, cia)

    for t in range(2):
        g = wid * 2 + t
        goff = g * 16

        ms = []
        is_ = []
        for _r in range(3):
            def fold_max(w, run):
                return jnp.maximum(run, cva[pl.ds(w * (_G * 16) + goff, 16)])
            runm = lax.fori_loop(0, _NW, fold_max,
                                 jnp.full((16,), -1.0, jnp.float32))
            mv = _bmax(runm)

            def fold_idx(w, run):
                off = w * (_G * 16) + goff
                return jnp.minimum(
                    run, jnp.where(cva[pl.ds(off, 16)] == mv,
                                   cia[pl.ds(off, 16)], _BIG))
            runi = lax.fori_loop(0, _NW, fold_idx,
                                 jnp.full((16,), _BIG, jnp.int32))
            iv = _bmin(runi)

            def mask_out(w, _):
                off = w * (_G * 16) + goff
                cva[pl.ds(off, 16)] = jnp.where(
                    cia[pl.ds(off, 16)] == iv, -1.0, cva[pl.ds(off, 16)])
                return 0
            lax.fori_loop(0, _NW, mask_out, 0)
            ms.append(mv)
            is_.append(iv)

        m1, m2, m3 = ms
        i1, i2, i3 = is_
        nposv = (jnp.where(m1 > _POS, 1, 0) + jnp.where(m2 > _POS, 1, 0)
                 + jnp.where(m3 > _POS, 1, 0))
        condv = nposv < 3
        l2 = jnp.where((m2 > _LOW) & condv, i2, -1)
        l3 = jnp.where((m3 > _LOW) & condv, i3, -1)
        wvr[...] = jnp.where(lane == 0, i1,
                             jnp.where(lane == 1, l2,
                                       jnp.where(lane == 2, l3, -1)))
        pltpu.sync_copy(wvr, writes_out.at[pl.ds(goff, 16)])


def _phase3(assign, writes, final_out, asgr, wvr):
    wid = _wid()
    base = wid * _PER_W
    lane = _lane()
    pltpu.sync_copy(assign.at[pl.ds(base, _PER_W)], asgr)
    pltpu.sync_copy(writes, wvr)

    def per_gt(g, _):
        wv = wvr[pl.ds(g * 16, 16)]
        t0 = _bcast(wv, 0)
        t1 = _bcast(wv, 1)
        t2 = _bcast(wv, 2)
        gvec = jnp.full((16,), g, jnp.int32)

        def per_chunk4(c4, _c):
            for u in range(4):
                off = c4 * 64 + u * 16
                a = asgr[pl.ds(off, 16)]
                gidx = (base + off) + lane
                upd = (gidx == t0) | (gidx == t1) | (gidx == t2)
                asgr[pl.ds(off, 16)] = jnp.where(upd, gvec, a)
            return 0
        lax.fori_loop(0, _CHUNKS // 4, per_chunk4, 0)
        return 0
    lax.fori_loop(0, _G, per_gt, 0)

    pltpu.sync_copy(asgr, final_out.at[pl.ds(base, _PER_W)])


@functools.lru_cache(maxsize=1)
def _build():
    mesh = plsc.VectorSubcoreMesh(core_axis_name="c", subcore_axis_name="s")
    f32, i32 = jnp.float32, jnp.int32
    p1 = pl.kernel(
        _phase1, mesh=mesh,
        out_type=(
            jax.ShapeDtypeStruct((_N_PAD,), i32),
            jax.ShapeDtypeStruct((_NW * _G * 16,), f32),
            jax.ShapeDtypeStruct((_NW * _G * 16,), i32),
        ),
        scratch_types=[
            pltpu.VMEM((_PER_W,), f32),
            pltpu.VMEM((_PER_W,), f32),
            pltpu.VMEM((_PER_W,), f32),
            pltpu.VMEM((_PER_W,), f32),
            pltpu.VMEM((_G * 16,), f32),
            pltpu.VMEM((_G * 16,), f32),
            pltpu.VMEM((_G * 16,), f32),
            pltpu.VMEM((_G * 16,), f32),
            pltpu.VMEM((_PER_W,), f32),
            pltpu.VMEM((_PER_W,), i32),
            pltpu.VMEM((_PER_W,), i32),
            pltpu.VMEM((_G * 16,), f32),
            pltpu.VMEM((_G * 16,), i32),
        ],
    )
    p2 = pl.kernel(
        _phase2, mesh=mesh,
        out_type=jax.ShapeDtypeStruct((_G * 16,), i32),
        scratch_types=[
            pltpu.VMEM((_NW * _G * 16,), f32),
            pltpu.VMEM((_NW * _G * 16,), i32),
            pltpu.VMEM((16,), i32),
        ],
    )
    p3 = pl.kernel(
        _phase3, mesh=mesh,
        out_type=jax.ShapeDtypeStruct((_N_PAD,), i32),
        scratch_types=[
            pltpu.VMEM((_PER_W,), i32),
            pltpu.VMEM((_G * 16,), i32),
        ],
    )
    return p1, p2, p3


def kernel(anchor, gt):
    n = anchor.shape[0]
    a = jnp.pad(anchor, ((0, _N_PAD - n), (0, 0)))
    gb = [jnp.repeat(gt[:, c], 16) for c in range(4)]
    p1, p2, p3 = _build()
    assign, candv, candi = p1(a[:, 0], a[:, 1], a[:, 2], a[:, 3], *gb)
    writes = p2(candv, candi)
    final = p3(assign, writes)
    return final[:n]


# speedup vs baseline: 1.1844x; 1.0847x over previous
import functools

import jax
import jax.numpy as jnp
from jax import lax
from jax.experimental import pallas as pl
from jax.experimental.pallas import tpu as pltpu
from jax.experimental.pallas import tpu_sc as plsc

_POS = 0.5
_NEG = 0.3
_LOW = 0.1
_N_PAD = 20480
_NW = 32
_PER_W = _N_PAD // _NW
_CHUNKS = _PER_W // 16
_G = 64
_BIG = 2**30


def _lane():
    return lax.broadcasted_iota(jnp.int32, (16,), 0)


def _bcast(x, l):
    idx = jnp.full((16,), l, jnp.int32)
    return x.at[idx].get(mode="promise_in_bounds")


def _bmax(x):
    for k in (1, 2, 4, 8):
        x = jnp.maximum(x, x.at[_lane() ^ k].get(mode="promise_in_bounds"))
    return x


def _bmin(x):
    for k in (1, 2, 4, 8):
        x = jnp.minimum(x, x.at[_lane() ^ k].get(mode="promise_in_bounds"))
    return x


def _wid():
    return lax.axis_index("s") * 2 + lax.axis_index("c")


def _phase1(ax0, ay0, ax1, ay1, gb0, gb1, gb2, gb3,
            assign_out, candv_out, candi_out,
            a0r, a1r, a2r, a3r, g0r, g1r, g2r, g3r,
            rmr, rir, asgr, cvr, cir):
    wid = _wid()
    base = wid * _PER_W
    pltpu.sync_copy(ax0.at[pl.ds(base, _PER_W)], a0r)
    pltpu.sync_copy(ay0.at[pl.ds(base, _PER_W)], a1r)
    pltpu.sync_copy(ax1.at[pl.ds(base, _PER_W)], a2r)
    pltpu.sync_copy(ay1.at[pl.ds(base, _PER_W)], a3r)
    pltpu.sync_copy(gb0, g0r)
    pltpu.sync_copy(gb1, g1r)
    pltpu.sync_copy(gb2, g2r)
    pltpu.sync_copy(gb3, g3r)

    lane = _lane()
    neg1f = jnp.full((16,), -1.0, jnp.float32)
    zeroi = jnp.full((16,), 0, jnp.int32)

    def init_c(c, _):
        rmr[pl.ds(c * 16, 16)] = neg1f
        rir[pl.ds(c * 16, 16)] = zeroi
        return 0
    lax.fori_loop(0, _CHUNKS, init_c, 0)

    def per_gt(g, _):
        goff = g * 16
        g0 = g0r[pl.ds(goff, 16)]
        g1 = g1r[pl.ds(goff, 16)]
        g2 = g2r[pl.ds(goff, 16)]
        g3 = g3r[pl.ds(goff, 16)]
        area_b = (g2 - g0) * (g3 - g1)
        gvec = jnp.full((16,), g, jnp.int32)

        def per_chunk4(c4, carry):
            t1, t2, t3, j1, j2, j3 = carry
            ious = []
            for u in range(4):
                off = c4 * 64 + u * 16
                a0 = a0r[pl.ds(off, 16)]
                a1 = a1r[pl.ds(off, 16)]
                a2 = a2r[pl.ds(off, 16)]
                a3 = a3r[pl.ds(off, 16)]
                area_a = (a2 - a0) * (a3 - a1)
                ltx = jnp.maximum(a0, g0)
                lty = jnp.maximum(a1, g1)
                rbx = jnp.minimum(a2, g2)
                rby = jnp.minimum(a3, g3)
                w = jnp.maximum(rbx - ltx, 0.0)
                h = jnp.maximum(rby - lty, 0.0)
                inter = w * h
                union = area_a + area_b - inter
                iou = inter / jnp.maximum(union, 1e-9)
                ious.append(iou)

                rm = rmr[pl.ds(off, 16)]
                upd = iou > rm
                rmr[pl.ds(off, 16)] = jnp.where(upd, iou, rm)
                rir[pl.ds(off, 16)] = jnp.where(upd, gvec, rir[pl.ds(off, 16)])

            for u in range(4):
                off = c4 * 64 + u * 16
                iou = ious[u]
                gidx = (base + off) + lane
                b1 = iou > t1
                b2 = iou > t2
                b3 = iou > t3
                t3n = jnp.where(b2, t2, jnp.where(b3, iou, t3))
                j3n = jnp.where(b2, j2, jnp.where(b3, gidx, j3))
                t2n = jnp.where(b1, t1, jnp.where(b2, iou, t2))
                j2n = jnp.where(b1, j1, jnp.where(b2, gidx, j2))
                t1, t2, t3 = jnp.where(b1, iou, t1), t2n, t3n
                j1, j2, j3 = jnp.where(b1, gidx, j1), j2n, j3n
            return t1, t2, t3, j1, j2, j3

        init = (neg1f, neg1f, neg1f, zeroi, zeroi, zeroi)
        t1, t2, t3, j1, j2, j3 = lax.fori_loop(0, _CHUNKS // 4, per_chunk4,
                                               init)

        d = zeroi
        m1 = _bmax(t1)
        i1 = _bmin(jnp.where(t1 == m1, j1, _BIG))
        d = d + jnp.where((t1 == m1) & (j1 == i1), 1, 0)
        ev = jnp.where(d == 1, t2, t1)
        ej = jnp.where(d == 1, j2, j1)
        m2 = _bmax(ev)
        i2 = _bmin(jnp.where(ev == m2, ej, _BIG))
        d = d + jnp.where((ev == m2) & (ej == i2), 1, 0)
        ev = jnp.where(d == 2, t3, jnp.where(d == 1, t2, t1))
        ej = jnp.where(d == 2, j3, jnp.where(d == 1, j2, j1))
        m3 = _bmax(ev)
        i3 = _bmin(jnp.where(ev == m3, ej, _BIG))

        cvr[g] = jnp.where(
            lane == 0, m1, jnp.where(lane == 1, m2,
                                     jnp.where(lane == 2, m3, -1.0)))
        cir[g] = jnp.where(
            lane == 0, i1, jnp.where(lane == 1, i2,
                                     jnp.where(lane == 2, i3, _BIG)))
        return 0

    lax.fori_loop(0, _G, per_gt, 0)

    def base_c(c, _):
        off = c * 16
        rm = rmr[pl.ds(off, 16)]
        a = jnp.where(rm > _POS, rir[pl.ds(off, 16)], -2)
        a = jnp.where(rm < _NEG, -1, a)
        asgr[pl.ds(off, 16)] = a
        return 0
    lax.fori_loop(0, _CHUNKS, base_c, 0)

    pltpu.sync_copy(asgr, assign_out.at[pl.ds(base, _PER_W)])
    pltpu.sync_copy(cvr, candv_out.at[wid])
    pltpu.sync_copy(cir, candi_out.at[wid])


def _phase2(candv, candi, writes_out, cva, cia, wvr):
    wid = _wid()
    lane = _lane()

    for t in range(2):
        g = wid * 2 + t
        goff = g * 16
        pltpu.sync_copy(candv.at[:, g], cva)
        pltpu.sync_copy(candi.at[:, g], cia)

        ms = []
        is_ = []
        for _r in range(3):
            def fold_max(w, run):
                return jnp.maximum(run, cva[w])
            runm = lax.fori_loop(0, _NW, fold_max,
                                 jnp.full((16,), -1.0, jnp.float32))
            mv = _bmax(runm)

            def fold_idx(w, run):
                return jnp.minimum(
                    run, jnp.where(cva[w] == mv, cia[w], _BIG))
            runi = lax.fori_loop(0, _NW, fold_idx,
                                 jnp.full((16,), _BIG, jnp.int32))
            iv = _bmin(runi)

            def mask_out(w, _):
                cva[w] = jnp.where(cia[w] == iv, -1.0, cva[w])
                return 0
            lax.fori_loop(0, _NW, mask_out, 0)
            ms.append(mv)
            is_.append(iv)

        m1, m2, m3 = ms
        i1, i2, i3 = is_
        nposv = (jnp.where(m1 > _POS, 1, 0) + jnp.where(m2 > _POS, 1, 0)
                 + jnp.where(m3 > _POS, 1, 0))
        condv = nposv < 3
        l2 = jnp.where((m2 > _LOW) & condv, i2, -1)
        l3 = jnp.where((m3 > _LOW) & condv, i3, -1)
        wvr[...] = jnp.where(lane == 0, i1,
                             jnp.where(lane == 1, l2,
                                       jnp.where(lane == 2, l3, -1)))
        pltpu.sync_copy(wvr, writes_out.at[pl.ds(goff, 16)])


def _phase3(assign, writes, final_out, asgr, wvr):
    wid = _wid()
    base = wid * _PER_W
    lane = _lane()
    pltpu.sync_copy(assign.at[pl.ds(base, _PER_W)], asgr)
    pltpu.sync_copy(writes, wvr)

    def per_gt(g, _):
        wv = wvr[pl.ds(g * 16, 16)]
        t0 = _bcast(wv, 0)
        t1 = _bcast(wv, 1)
        t2 = _bcast(wv, 2)
        gvec = jnp.full((16,), g, jnp.int32)

        def per_chunk4(c4, _c):
            for u in range(4):
                off = c4 * 64 + u * 16
                a = asgr[pl.ds(off, 16)]
                gidx = (base + off) + lane
                upd = (gidx == t0) | (gidx == t1) | (gidx == t2)
                asgr[pl.ds(off, 16)] = jnp.where(upd, gvec, a)
            return 0
        lax.fori_loop(0, _CHUNKS // 4, per_chunk4, 0)
        return 0
    lax.fori_loop(0, _G, per_gt, 0)

    pltpu.sync_copy(asgr, final_out.at[pl.ds(base, _PER_W)])


@functools.lru_cache(maxsize=1)
def _build():
    mesh = plsc.VectorSubcoreMesh(core_axis_name="c", subcore_axis_name="s")
    f32, i32 = jnp.float32, jnp.int32
    p1 = pl.kernel(
        _phase1, mesh=mesh,
        out_type=(
            jax.ShapeDtypeStruct((_N_PAD,), i32),
            jax.ShapeDtypeStruct((_NW, _G, 16), f32),
            jax.ShapeDtypeStruct((_NW, _G, 16), i32),
        ),
        scratch_types=[
            pltpu.VMEM((_PER_W,), f32),
            pltpu.VMEM((_PER_W,), f32),
            pltpu.VMEM((_PER_W,), f32),
            pltpu.VMEM((_PER_W,), f32),
            pltpu.VMEM((_G * 16,), f32),
            pltpu.VMEM((_G * 16,), f32),
            pltpu.VMEM((_G * 16,), f32),
            pltpu.VMEM((_G * 16,), f32),
            pltpu.VMEM((_PER_W,), f32),
            pltpu.VMEM((_PER_W,), i32),
            pltpu.VMEM((_PER_W,), i32),
            pltpu.VMEM((_G, 16), f32),
            pltpu.VMEM((_G, 16), i32),
        ],
    )
    p2 = pl.kernel(
        _phase2, mesh=mesh,
        out_type=jax.ShapeDtypeStruct((_G * 16,), i32),
        scratch_types=[
            pltpu.VMEM((_NW, 16), f32),
            pltpu.VMEM((_NW, 16), i32),
            pltpu.VMEM((16,), i32),
        ],
    )
    p3 = pl.kernel(
        _phase3, mesh=mesh,
        out_type=jax.ShapeDtypeStruct((_N_PAD,), i32),
        scratch_types=[
            pltpu.VMEM((_PER_W,), i32),
            pltpu.VMEM((_G * 16,), i32),
        ],
    )
    return p1, p2, p3


def kernel(anchor, gt):
    n = anchor.shape[0]
    a = jnp.pad(anchor, ((0, _N_PAD - n), (0, 0)))
    gb = [jnp.repeat(gt[:, c], 16) for c in range(4)]
    p1, p2, p3 = _build()
    assign, candv, candi = p1(a[:, 0], a[:, 1], a[:, 2], a[:, 3], *gb)
    writes = p2(candv, candi)
    final = p3(assign, writes)
    return final[:n]


# speedup vs baseline: 1.1981x; 1.0115x over previous
import functools

import jax
import jax.numpy as jnp
from jax import lax
from jax.experimental import pallas as pl
from jax.experimental.pallas import tpu as pltpu
from jax.experimental.pallas import tpu_sc as plsc

_POS = 0.5
_NEG = 0.3
_LOW = 0.1
_N_PAD = 20480
_NW = 32
_PER_W = _N_PAD // _NW
_CHUNKS = _PER_W // 16
_G = 64
_BIG = 2**30


def _lane():
    return lax.broadcasted_iota(jnp.int32, (16,), 0)


def _bcast(x, l):
    idx = jnp.full((16,), l, jnp.int32)
    return x.at[idx].get(mode="promise_in_bounds")


def _bmax(x):
    for k in (1, 2, 4, 8):
        x = jnp.maximum(x, x.at[_lane() ^ k].get(mode="promise_in_bounds"))
    return x


def _bmin(x):
    for k in (1, 2, 4, 8):
        x = jnp.minimum(x, x.at[_lane() ^ k].get(mode="promise_in_bounds"))
    return x


def _wid():
    return lax.axis_index("s") * 2 + lax.axis_index("c")


def _phase1(ax0, ay0, ax1, ay1, gb0, gb1, gb2, gb3,
            assign_out, candv_out, candi_out,
            a0r, a1r, a2r, a3r, g0r, g1r, g2r, g3r,
            rmr, rir, asgr, cvr, cir, aar):
    wid = _wid()
    base = wid * _PER_W
    pltpu.sync_copy(ax0.at[pl.ds(base, _PER_W)], a0r)
    pltpu.sync_copy(ay0.at[pl.ds(base, _PER_W)], a1r)
    pltpu.sync_copy(ax1.at[pl.ds(base, _PER_W)], a2r)
    pltpu.sync_copy(ay1.at[pl.ds(base, _PER_W)], a3r)
    pltpu.sync_copy(gb0, g0r)
    pltpu.sync_copy(gb1, g1r)
    pltpu.sync_copy(gb2, g2r)
    pltpu.sync_copy(gb3, g3r)

    lane = _lane()
    neg1f = jnp.full((16,), -1.0, jnp.float32)
    zeroi = jnp.full((16,), 0, jnp.int32)

    def init_c(c, _):
        off = c * 16
        rmr[pl.ds(off, 16)] = neg1f
        rir[pl.ds(off, 16)] = zeroi
        a0 = a0r[pl.ds(off, 16)]
        a1 = a1r[pl.ds(off, 16)]
        a2 = a2r[pl.ds(off, 16)]
        a3 = a3r[pl.ds(off, 16)]
        aar[pl.ds(off, 16)] = (a2 - a0) * (a3 - a1)
        return 0
    lax.fori_loop(0, _CHUNKS, init_c, 0)

    def per_gt(g, _):
        goff = g * 16
        g0 = g0r[pl.ds(goff, 16)]
        g1 = g1r[pl.ds(goff, 16)]
        g2 = g2r[pl.ds(goff, 16)]
        g3 = g3r[pl.ds(goff, 16)]
        area_b = (g2 - g0) * (g3 - g1)
        gvec = jnp.full((16,), g, jnp.int32)

        def per_chunk4(c4, carry):
            t1, t2, t3, j1, j2, j3 = carry
            ious = []
            for u in range(4):
                off = c4 * 64 + u * 16
                a0 = a0r[pl.ds(off, 16)]
                a1 = a1r[pl.ds(off, 16)]
                a2 = a2r[pl.ds(off, 16)]
                a3 = a3r[pl.ds(off, 16)]
                area_a = aar[pl.ds(off, 16)]
                ltx = jnp.maximum(a0, g0)
                lty = jnp.maximum(a1, g1)
                rbx = jnp.minimum(a2, g2)
                rby = jnp.minimum(a3, g3)
                w = jnp.maximum(rbx - ltx, 0.0)
                h = jnp.maximum(rby - lty, 0.0)
                inter = w * h
                union = area_a + area_b - inter
                iou = inter / jnp.maximum(union, 1e-9)
                ious.append(iou)

                rm = rmr[pl.ds(off, 16)]
                upd = iou > rm
                rmr[pl.ds(off, 16)] = jnp.where(upd, iou, rm)
                rir[pl.ds(off, 16)] = jnp.where(upd, gvec, rir[pl.ds(off, 16)])

            for u in range(4):
                off = c4 * 64 + u * 16
                iou = ious[u]
                gidx = (base + off) + lane
                b1 = iou > t1
                b2 = iou > t2
                b3 = iou > t3
                t3n = jnp.where(b2, t2, jnp.where(b3, iou, t3))
                j3n = jnp.where(b2, j2, jnp.where(b3, gidx, j3))
                t2n = jnp.where(b1, t1, jnp.where(b2, iou, t2))
                j2n = jnp.where(b1, j1, jnp.where(b2, gidx, j2))
                t1, t2, t3 = jnp.where(b1, iou, t1), t2n, t3n
                j1, j2, j3 = jnp.where(b1, gidx, j1), j2n, j3n
            return t1, t2, t3, j1, j2, j3

        init = (neg1f, neg1f, neg1f, zeroi, zeroi, zeroi)
        t1, t2, t3, j1, j2, j3 = lax.fori_loop(0, _CHUNKS // 4, per_chunk4,
                                               init)

        d = zeroi
        m1 = _bmax(t1)
        i1 = _bmin(jnp.where(t1 == m1, j1, _BIG))
        d = d + jnp.where((t1 == m1) & (j1 == i1), 1, 0)
        ev = jnp.where(d == 1, t2, t1)
        ej = jnp.where(d == 1, j2, j1)
        m2 = _bmax(ev)
        i2 = _bmin(jnp.where(ev == m2, ej, _BIG))
        d = d + jnp.where((ev == m2) & (ej == i2), 1, 0)
        ev = jnp.where(d == 2, t3, jnp.where(d == 1, t2, t1))
        ej = jnp.where(d == 2, j3, jnp.where(d == 1, j2, j1))
        m3 = _bmax(ev)
        i3 = _bmin(jnp.where(ev == m3, ej, _BIG))

        cvr[g] = jnp.where(
            lane == 0, m1, jnp.where(lane == 1, m2,
                                     jnp.where(lane == 2, m3, -1.0)))
        cir[g] = jnp.where(
            lane == 0, i1, jnp.where(lane == 1, i2,
                                     jnp.where(lane == 2, i3, _BIG)))
        return 0

    lax.fori_loop(0, _G, per_gt, 0)

    def base_c(c, _):
        off = c * 16
        rm = rmr[pl.ds(off, 16)]
        a = jnp.where(rm > _POS, rir[pl.ds(off, 16)], -2)
        a = jnp.where(rm < _NEG, -1, a)
        asgr[pl.ds(off, 16)] = a
        return 0
    lax.fori_loop(0, _CHUNKS, base_c, 0)

    pltpu.sync_copy(asgr, assign_out.at[pl.ds(base, _PER_W)])
    pltpu.sync_copy(cvr, candv_out.at[wid])
    pltpu.sync_copy(cir, candi_out.at[wid])


def _phase2(candv, candi, writes_out, cva, cia, wvr):
    wid = _wid()
    lane = _lane()

    for t in range(2):
        g = wid * 2 + t
        goff = g * 16
        pltpu.sync_copy(candv.at[:, g], cva)
        pltpu.sync_copy(candi.at[:, g], cia)

        ms = []
        is_ = []
        for _r in range(3):
            def fold_max(w, run):
                return jnp.maximum(run, cva[w])
            runm = lax.fori_loop(0, _NW, fold_max,
                                 jnp.full((16,), -1.0, jnp.float32))
            mv = _bmax(runm)

            def fold_idx(w, run):
                return jnp.minimum(
                    run, jnp.where(cva[w] == mv, cia[w], _BIG))
            runi = lax.fori_loop(0, _NW, fold_idx,
                                 jnp.full((16,), _BIG, jnp.int32))
            iv = _bmin(runi)

            def mask_out(w, _):
                cva[w] = jnp.where(cia[w] == iv, -1.0, cva[w])
                return 0
            lax.fori_loop(0, _NW, mask_out, 0)
            ms.append(mv)
            is_.append(iv)

        m1, m2, m3 = ms
        i1, i2, i3 = is_
        nposv = (jnp.where(m1 > _POS, 1, 0) + jnp.where(m2 > _POS, 1, 0)
                 + jnp.where(m3 > _POS, 1, 0))
        condv = nposv < 3
        l2 = jnp.where((m2 > _LOW) & condv, i2, -1)
        l3 = jnp.where((m3 > _LOW) & condv, i3, -1)
        wvr[...] = jnp.where(lane == 0, i1,
                             jnp.where(lane == 1, l2,
                                       jnp.where(lane == 2, l3, -1)))
        pltpu.sync_copy(wvr, writes_out.at[pl.ds(goff, 16)])


def _phase3(assign, writes, final_out, asgr, wvr):
    wid = _wid()
    base = wid * _PER_W
    lane = _lane()
    pltpu.sync_copy(assign.at[pl.ds(base, _PER_W)], asgr)
    pltpu.sync_copy(writes, wvr)

    def per_gt(g, _):
        wv = wvr[pl.ds(g * 16, 16)]
        t0 = _bcast(wv, 0)
        t1 = _bcast(wv, 1)
        t2 = _bcast(wv, 2)
        gvec = jnp.full((16,), g, jnp.int32)

        def per_chunk4(c4, _c):
            for u in range(4):
                off = c4 * 64 + u * 16
                a = asgr[pl.ds(off, 16)]
                gidx = (base + off) + lane
                upd = (gidx == t0) | (gidx == t1) | (gidx == t2)
                asgr[pl.ds(off, 16)] = jnp.where(upd, gvec, a)
            return 0
        lax.fori_loop(0, _CHUNKS // 4, per_chunk4, 0)
        return 0
    lax.fori_loop(0, _G, per_gt, 0)

    pltpu.sync_copy(asgr, final_out.at[pl.ds(base, _PER_W)])


@functools.lru_cache(maxsize=1)
def _build():
    mesh = plsc.VectorSubcoreMesh(core_axis_name="c", subcore_axis_name="s")
    f32, i32 = jnp.float32, jnp.int32
    p1 = pl.kernel(
        _phase1, mesh=mesh,
        out_type=(
            jax.ShapeDtypeStruct((_N_PAD,), i32),
            jax.ShapeDtypeStruct((_NW, _G, 16), f32),
            jax.ShapeDtypeStruct((_NW, _G, 16), i32),
        ),
        scratch_types=[
            pltpu.VMEM((_PER_W,), f32),
            pltpu.VMEM((_PER_W,), f32),
            pltpu.VMEM((_PER_W,), f32),
            pltpu.VMEM((_PER_W,), f32),
            pltpu.VMEM((_G * 16,), f32),
            pltpu.VMEM((_G * 16,), f32),
            pltpu.VMEM((_G * 16,), f32),
            pltpu.VMEM((_G * 16,), f32),
            pltpu.VMEM((_PER_W,), f32),
            pltpu.VMEM((_PER_W,), i32),
            pltpu.VMEM((_PER_W,), i32),
            pltpu.VMEM((_G, 16), f32),
            pltpu.VMEM((_G, 16), i32),
            pltpu.VMEM((_PER_W,), f32),
        ],
    )
    p2 = pl.kernel(
        _phase2, mesh=mesh,
        out_type=jax.ShapeDtypeStruct((_G * 16,), i32),
        scratch_types=[
            pltpu.VMEM((_NW, 16), f32),
            pltpu.VMEM((_NW, 16), i32),
            pltpu.VMEM((16,), i32),
        ],
    )
    p3 = pl.kernel(
        _phase3, mesh=mesh,
        out_type=jax.ShapeDtypeStruct((_N_PAD,), i32),
        scratch_types=[
            pltpu.VMEM((_PER_W,), i32),
            pltpu.VMEM((_G * 16,), i32),
        ],
    )
    return p1, p2, p3


def kernel(anchor, gt):
    n = anchor.shape[0]
    a = jnp.pad(anchor, ((0, _N_PAD - n), (0, 0)))
    gb = [jnp.repeat(gt[:, c], 16) for c in range(4)]
    p1, p2, p3 = _build()
    assign, candv, candi = p1(a[:, 0], a[:, 1], a[:, 2], a[:, 3], *gb)
    writes = p2(candv, candi)
    final = p3(assign, writes)
    return final[:n]
